# baseline (speedup 1.0000x reference)
import functools
import jax
import jax.numpy as jnp
from jax import lax
from jax.experimental import pallas as pl
from jax.experimental.pallas import tpu as pltpu
from jax.experimental.pallas import tpu_sc as plsc

B, S, C = 4, 8192, 2048
CPW = 256
R = 64
NCH = S // R
NJ = CPW // 16

_mesh = plsc.VectorSubcoreMesh(core_axis_name="c", subcore_axis_name="s")


@functools.partial(
    pl.kernel,
    mesh=_mesh,
    out_type=jax.ShapeDtypeStruct((B, S, C), jnp.float32),
    scratch_types=[
        pltpu.VMEM((R, CPW), jnp.float32),
        pltpu.VMEM((R, CPW), jnp.float32),
        pltpu.VMEM((NJ, 16), jnp.float32),
        pltpu.SemaphoreType.DMA,
        pltpu.SemaphoreType.DMA,
    ],
)
def _sc_scan(x_hbm, o_hbm, in_v, out_v, carry_v, sem_in, sem_out):
    wid = lax.axis_index("s") * 2 + lax.axis_index("c")
    b = wid // 8
    c0 = (wid % 8) * CPW

    for j in range(NJ):
        carry_v[j] = jnp.zeros((16,), jnp.float32)

    def chunk(i, _):
        r0 = i * R
        pltpu.async_copy(
            x_hbm.at[b, pl.ds(r0, R), pl.ds(c0, CPW)], in_v, sem_in
        ).wait()

        def row(r, _):
            for j in range(NJ):
                v = in_v[r, pl.ds(16 * j, 16)] + carry_v[j]
                out_v[r, pl.ds(16 * j, 16)] = v
                carry_v[j] = v
            return 0

        lax.fori_loop(0, R, row, 0)
        pltpu.async_copy(
            out_v, o_hbm.at[b, pl.ds(r0, R), pl.ds(c0, CPW)], sem_out
        ).wait()
        return 0

    lax.fori_loop(0, NCH, chunk, 0)


def kernel(x):
    return _sc_scan(x)


# speedup vs baseline: 2.6458x; 2.6458x over previous
import functools
import jax
import jax.numpy as jnp
from jax import lax
from jax.experimental import pallas as pl
from jax.experimental.pallas import tpu as pltpu
from jax.experimental.pallas import tpu_sc as plsc

B, S, C = 4, 8192, 2048
CPW = 256
R = 64
NCH = S // R
NJ = CPW // 16

_mesh = plsc.VectorSubcoreMesh(core_axis_name="c", subcore_axis_name="s")


@functools.partial(
    pl.kernel,
    mesh=_mesh,
    out_type=jax.ShapeDtypeStruct((B, S, C), jnp.float32),
    scratch_types=[
        pltpu.VMEM((R, CPW), jnp.float32),
        pltpu.VMEM((R, CPW), jnp.float32),
        pltpu.SemaphoreType.DMA,
        pltpu.SemaphoreType.DMA,
    ],
)
def _sc_scan(x_hbm, o_hbm, in_v, out_v, sem_in, sem_out):
    wid = lax.axis_index("s") * 2 + lax.axis_index("c")
    b = wid // 8
    c0 = (wid % 8) * CPW

    def chunk(i, carries):
        r0 = i * R
        pltpu.async_copy(
            x_hbm.at[b, pl.ds(r0, R), pl.ds(c0, CPW)], in_v, sem_in
        ).wait()

        def row(r, cs):
            new = []
            for j in range(NJ):
                v = in_v[r, pl.ds(16 * j, 16)] + cs[j]
                out_v[r, pl.ds(16 * j, 16)] = v
                new.append(v)
            return tuple(new)

        carries = lax.fori_loop(0, R, row, carries)
        pltpu.async_copy(
            out_v, o_hbm.at[b, pl.ds(r0, R), pl.ds(c0, CPW)], sem_out
        ).wait()
        return carries

    zeros = tuple(jnp.zeros((16,), jnp.float32) for _ in range(NJ))
    lax.fori_loop(0, NCH, chunk, zeros)


def kernel(x):
    return _sc_scan(x)


# speedup vs baseline: 4.7429x; 1.7926x over previous
import functools
import jax
import jax.numpy as jnp
from jax import lax
from jax.experimental import pallas as pl
from jax.experimental.pallas import tpu as pltpu
from jax.experimental.pallas import tpu_sc as plsc

B, S, C = 4, 8192, 2048
CPW = 256
R = 64
NCH = S // R
NJ = CPW // 16

_mesh = plsc.VectorSubcoreMesh(core_axis_name="c", subcore_axis_name="s")


@functools.partial(
    pl.kernel,
    mesh=_mesh,
    out_type=jax.ShapeDtypeStruct((B, S, C), jnp.float32),
    scratch_types=[
        pltpu.VMEM((R, CPW), jnp.float32),
        pltpu.VMEM((R, CPW), jnp.float32),
        pltpu.VMEM((R, CPW), jnp.float32),
        pltpu.VMEM((R, CPW), jnp.float32),
        pltpu.SemaphoreType.DMA,
        pltpu.SemaphoreType.DMA,
        pltpu.SemaphoreType.DMA,
        pltpu.SemaphoreType.DMA,
    ],
)
def _sc_scan(x_hbm, o_hbm, in0, in1, out0, out1, si0, si1, so0, so1):
    wid = lax.axis_index("s") * 2 + lax.axis_index("c")
    b = wid // 8
    c0 = (wid % 8) * CPW

    def src(i):
        return x_hbm.at[b, pl.ds(i * R, R), pl.ds(c0, CPW)]

    def dst(i):
        return o_hbm.at[b, pl.ds(i * R, R), pl.ds(c0, CPW)]

    def compute_rows(in_b, out_b, cs):
        def row(r, cs):
            new = []
            for j in range(NJ):
                v = in_b[r, pl.ds(16 * j, 16)] + cs[j]
                out_b[r, pl.ds(16 * j, 16)] = v
                new.append(v)
            return tuple(new)

        return lax.fori_loop(0, R, row, cs)

    pltpu.async_copy(src(0), in0, si0)
    pltpu.async_copy(src(1), in1, si1)
    cs = tuple(jnp.zeros((16,), jnp.float32) for _ in range(NJ))

    pltpu.make_async_copy(src(0), in0, si0).wait()
    cs = compute_rows(in0, out0, cs)
    pltpu.async_copy(out0, dst(0), so0)
    pltpu.async_copy(src(2), in0, si0)

    pltpu.make_async_copy(src(1), in1, si1).wait()
    cs = compute_rows(in1, out1, cs)
    pltpu.async_copy(out1, dst(1), so1)
    pltpu.async_copy(src(3), in1, si1)

    def pair(k, cs):
        i0 = 2 * k
        i1 = i0 + 1

        pltpu.make_async_copy(out0, dst(i0 - 2), so0).wait()
        pltpu.make_async_copy(src(i0), in0, si0).wait()
        cs = compute_rows(in0, out0, cs)
        pltpu.async_copy(out0, dst(i0), so0)

        @pl.when(i0 + 2 < NCH)
        def _():
            pltpu.async_copy(src(i0 + 2), in0, si0)

        pltpu.make_async_copy(out1, dst(i1 - 2), so1).wait()
        pltpu.make_async_copy(src(i1), in1, si1).wait()
        cs = compute_rows(in1, out1, cs)
        pltpu.async_copy(out1, dst(i1), so1)

        @pl.when(i1 + 2 < NCH)
        def _():
            pltpu.async_copy(src(i1 + 2), in1, si1)

        return cs

    lax.fori_loop(1, NCH // 2, pair, cs)
    pltpu.make_async_copy(out0, dst(NCH - 2), so0).wait()
    pltpu.make_async_copy(out1, dst(NCH - 1), so1).wait()


def kernel(x):
    return _sc_scan(x)
